# baseline (speedup 1.0000x reference)
import jax
import jax.numpy as jnp
from jax import lax
from jax.experimental import pallas as pl
from jax.experimental.pallas import tpu as pltpu
from jax.experimental.pallas import tpu_sc as plsc

VOCAB = 100000
DIM = 128
NUM_TOKENS = 20
BATCH = 1024
SEQ = 220
CHUNK = 256
CPP = BATCH // CHUNK
NCHUNKS = SEQ * CPP
NBUF = 3

_info = plsc.get_sparse_core_info()
_NC, _NS = _info.num_cores, _info.num_subcores
NW = _NC * _NS
STEPS = -(-NCHUNKS // NW)


def _sc_body(tok_hbm, table_hbm, prompt_hbm, out_hbm,
             idx_all, buf0, buf1, buf2,
             sem_i, sem_g0, sem_g1, sem_g2, sem_w0, sem_w1, sem_w2):
    wid = lax.axis_index("s") * _NC + lax.axis_index("c")

    bufs = (buf0, buf1, buf2)
    sems_g = (sem_g0, sem_g1, sem_g2)
    sems_w = (sem_w0, sem_w1, sem_w2)

    def chunk_of(s):
        t = wid + s * NW
        t = jnp.where(t < NCHUNKS, t, wid)
        return t // CPP, t % CPP

    jcs = [chunk_of(s) for s in range(STEPS)]
    pend_i = []
    for s, (j, c) in enumerate(jcs):
        off = jnp.where(j < NUM_TOKENS, 0, j * BATCH + c * CHUNK)
        pend_i.append(pltpu.async_copy(tok_hbm.at[pl.ds(off, CHUNK)],
                                       idx_all.at[pl.ds(s * CHUNK, CHUNK)],
                                       sem_i))
    for d in pend_i:
        d.wait()
    for s, (j, c) in enumerate(jcs):
        @pl.when(j < NUM_TOKENS)
        def _(j=j, s=s):
            fill = jnp.broadcast_to(j.astype(jnp.int32), (16,))
            for q in range(CHUNK // 16):
                idx_all[pl.ds(s * CHUNK + q * 16, 16)] = fill

    def issue(s, k):
        j, c = jcs[s]
        idx = idx_all.at[pl.ds(s * CHUNK, CHUNK)]

        @pl.when(j < NUM_TOKENS)
        def _():
            pltpu.async_copy(prompt_hbm.at[idx], bufs[k], sems_g[k])

        @pl.when(j >= NUM_TOKENS)
        def _():
            pltpu.async_copy(table_hbm.at[idx], bufs[k], sems_g[k])

    def wait_gather(s, k):
        idx = idx_all.at[pl.ds(s * CHUNK, CHUNK)]
        pltpu.make_async_copy(table_hbm.at[idx], bufs[k], sems_g[k]).wait()

    pend_w = [None] * NBUF
    for s in range(NBUF - 1):
        issue(s, s)
    for s in range(STEPS):
        k = s % NBUF
        nk = (s + NBUF - 1) % NBUF
        if s + NBUF - 1 < STEPS:
            if pend_w[nk] is not None:
                pend_w[nk].wait()
                pend_w[nk] = None
            issue(s + NBUF - 1, nk)
        wait_gather(s, k)
        j, c = jcs[s]
        pend_w[k] = pltpu.async_copy(bufs[k], out_hbm.at[j, pl.ds(c * CHUNK, CHUNK)],
                                     sems_w[k])
    for k in range(NBUF):
        if pend_w[k] is not None:
            pend_w[k].wait()


def kernel(tokens, table, prompt_embedding):
    tok = jnp.transpose(tokens.astype(jnp.int32)).reshape(-1)
    sc = pl.kernel(
        _sc_body,
        out_type=jax.ShapeDtypeStruct((SEQ, BATCH, DIM), jnp.float32),
        mesh=plsc.VectorSubcoreMesh(core_axis_name="c", subcore_axis_name="s"),
        scratch_types=[
            pltpu.VMEM((STEPS * CHUNK,), jnp.int32),
            pltpu.VMEM((CHUNK, DIM), jnp.float32),
            pltpu.VMEM((CHUNK, DIM), jnp.float32),
            pltpu.VMEM((CHUNK, DIM), jnp.float32),
            pltpu.SemaphoreType.DMA,
            pltpu.SemaphoreType.DMA,
            pltpu.SemaphoreType.DMA,
            pltpu.SemaphoreType.DMA,
            pltpu.SemaphoreType.DMA,
            pltpu.SemaphoreType.DMA,
            pltpu.SemaphoreType.DMA,
        ],
    )
    out = sc(tok, table, prompt_embedding)
    return jnp.transpose(out, (1, 0, 2))


# speedup vs baseline: 1.9765x; 1.9765x over previous
import jax
import jax.numpy as jnp
from jax import lax
from jax.experimental import pallas as pl
from jax.experimental.pallas import tpu as pltpu
from jax.experimental.pallas import tpu_sc as plsc

VOCAB = 100000
DIM = 128
NUM_TOKENS = 20
BATCH = 1024
SEQ = 220
BODY = SEQ - NUM_TOKENS
CHUNK = 256
CPP = BATCH // CHUNK
NBODY = BODY * CPP
NBUF = 3

_info = plsc.get_sparse_core_info()
_NC, _NS = _info.num_cores, _info.num_subcores
NW = _NC * _NS
STEPS = NBODY // NW


def _sc_body(tok_hbm, table_hbm, prompt_hbm, out_hbm,
             idx_all, buf0, buf1, buf2,
             sem_i, sem_g0, sem_g1, sem_g2, sem_w0, sem_w1, sem_w2):
    wid = lax.axis_index("s") * _NC + lax.axis_index("c")

    bufs = (buf0, buf1, buf2)
    sems_g = (sem_g0, sem_g1, sem_g2)
    sems_w = (sem_w0, sem_w1, sem_w2)

    def chunk_of(s):
        t = wid + s * NW
        return NUM_TOKENS + t // CPP, t % CPP

    prompt_writes = []

    REP = 64

    @pl.when(wid < NUM_TOKENS)
    def _():
        pltpu.sync_copy(prompt_hbm.at[pl.ds(wid, 1)], buf0.at[pl.ds(0, 1)])
        vs = [buf0[0, pl.ds(q * 16, 16)] for q in range(DIM // 16)]

        def rep_body(r, carry):
            for q in range(DIM // 16):
                buf0[r, pl.ds(q * 16, 16)] = vs[q]
            return carry

        lax.fori_loop(1, REP, rep_body, 0)
        for h in range(BATCH // REP):
            pltpu.async_copy(buf0.at[pl.ds(0, REP)],
                             out_hbm.at[wid, pl.ds(h * REP, REP)], sem_w0)

    def drain_prompt_writes():
        for h in range(BATCH // REP):
            @pl.when(wid < NUM_TOKENS)
            def _(h=h):
                pltpu.make_async_copy(
                    buf0.at[pl.ds(0, REP)],
                    out_hbm.at[wid, pl.ds(h * REP, REP)], sem_w0
                ).wait()

    jcs = [chunk_of(s) for s in range(STEPS)]
    pend_i = []
    for s, (j, c) in enumerate(jcs):
        pend_i.append(pltpu.async_copy(
            tok_hbm.at[pl.ds(j * BATCH + c * CHUNK, CHUNK)],
            idx_all.at[pl.ds(s * CHUNK, CHUNK)], sem_i))

    def issue(s, k):
        idx = idx_all.at[pl.ds(s * CHUNK, CHUNK)]
        pend_i[s].wait()
        pltpu.async_copy(table_hbm.at[idx], bufs[k], sems_g[k])

    def wait_gather(s, k):
        idx = idx_all.at[pl.ds(s * CHUNK, CHUNK)]
        pltpu.make_async_copy(table_hbm.at[idx], bufs[k], sems_g[k]).wait()

    order = [(s + 1) % NBUF for s in range(STEPS)]
    pend_w = [None] * NBUF
    issue(0, order[0])
    issue(1, order[1])
    drained_prompt = [False]
    for s in range(STEPS):
        k = order[s]
        if s + NBUF - 1 < STEPS:
            nk = order[s + NBUF - 1]
            if nk == 0 and not drained_prompt[0]:
                drain_prompt_writes()
                drained_prompt[0] = True
            if pend_w[nk] is not None:
                pend_w[nk].wait()
                pend_w[nk] = None
            issue(s + NBUF - 1, nk)
        wait_gather(s, k)
        j, c = jcs[s]
        pend_w[k] = pltpu.async_copy(bufs[k], out_hbm.at[j, pl.ds(c * CHUNK, CHUNK)],
                                     sems_w[k])
    for k in range(NBUF):
        if pend_w[k] is not None:
            pend_w[k].wait()


def kernel(tokens, table, prompt_embedding):
    tok = jnp.transpose(tokens.astype(jnp.int32)).reshape(-1)
    sc = pl.kernel(
        _sc_body,
        out_type=jax.ShapeDtypeStruct((SEQ, BATCH, DIM), jnp.float32),
        mesh=plsc.VectorSubcoreMesh(core_axis_name="c", subcore_axis_name="s"),
        scratch_types=[
            pltpu.VMEM((STEPS * CHUNK,), jnp.int32),
            pltpu.VMEM((CHUNK, DIM), jnp.float32),
            pltpu.VMEM((CHUNK, DIM), jnp.float32),
            pltpu.VMEM((CHUNK, DIM), jnp.float32),
            pltpu.SemaphoreType.DMA,
            pltpu.SemaphoreType.DMA,
            pltpu.SemaphoreType.DMA,
            pltpu.SemaphoreType.DMA,
            pltpu.SemaphoreType.DMA,
            pltpu.SemaphoreType.DMA,
            pltpu.SemaphoreType.DMA,
        ],
    )
    out = sc(tok, table, prompt_embedding)
    return jnp.transpose(out, (1, 0, 2))


# speedup vs baseline: 1.9776x; 1.0005x over previous
import jax
import jax.numpy as jnp
from jax import lax
from jax.experimental import pallas as pl
from jax.experimental.pallas import tpu as pltpu
from jax.experimental.pallas import tpu_sc as plsc

VOCAB = 100000
DIM = 128
NUM_TOKENS = 20
BATCH = 1024
SEQ = 220
BODY = SEQ - NUM_TOKENS
CHUNK = 256
CPP = BATCH // CHUNK
NBODY = BODY * CPP
NBUF = 3

_info = plsc.get_sparse_core_info()
_NC, _NS = _info.num_cores, _info.num_subcores
NW = _NC * _NS
STEPS = NBODY // NW


def _sc_body(tok_hbm, table_hbm, prompt_hbm, out_hbm,
             idx_all, buf0, buf1, buf2,
             sem_i, sem_g0, sem_g1, sem_g2, sem_w0, sem_w1, sem_w2):
    wid = lax.axis_index("s") * _NC + lax.axis_index("c")

    bufs = (buf0, buf1, buf2)
    sems_g = (sem_g0, sem_g1, sem_g2)
    sems_w = (sem_w0, sem_w1, sem_w2)

    def chunk_of(s):
        t = wid + s * NW
        return NUM_TOKENS + t // CPP, t % CPP

    prompt_writes = []

    REP = 64

    @pl.when(wid < NUM_TOKENS)
    def _():
        pltpu.sync_copy(prompt_hbm.at[pl.ds(wid, 1)], buf0.at[pl.ds(0, 1)])
        vs = [buf0[0, pl.ds(q * 16, 16)] for q in range(DIM // 16)]

        def rep_body(r, carry):
            for q in range(DIM // 16):
                buf0[r, pl.ds(q * 16, 16)] = vs[q]
            return carry

        lax.fori_loop(1, REP, rep_body, 0)
        for h in range(BATCH // REP):
            pltpu.async_copy(buf0.at[pl.ds(0, REP)],
                             out_hbm.at[wid, pl.ds(h * REP, REP)], sem_w0)

    def drain_prompt_writes():
        for h in range(BATCH // REP):
            @pl.when(wid < NUM_TOKENS)
            def _(h=h):
                pltpu.make_async_copy(
                    buf0.at[pl.ds(0, REP)],
                    out_hbm.at[wid, pl.ds(h * REP, REP)], sem_w0
                ).wait()

    jcs = [chunk_of(s) for s in range(STEPS)]
    pend_i = []
    for s, (j, c) in enumerate(jcs):
        pend_i.append(pltpu.async_copy(
            tok_hbm.at[j, pl.ds(c * CHUNK, CHUNK)],
            idx_all.at[pl.ds(s * CHUNK, CHUNK)], sem_i))

    def issue(s, k):
        idx = idx_all.at[pl.ds(s * CHUNK, CHUNK)]
        pend_i[s].wait()
        pltpu.async_copy(table_hbm.at[idx], bufs[k], sems_g[k])

    def wait_gather(s, k):
        idx = idx_all.at[pl.ds(s * CHUNK, CHUNK)]
        pltpu.make_async_copy(table_hbm.at[idx], bufs[k], sems_g[k]).wait()

    order = [(s + 1) % NBUF for s in range(STEPS)]
    pend_w = [None] * NBUF
    issue(0, order[0])
    issue(1, order[1])
    drained_prompt = [False]
    for s in range(STEPS):
        k = order[s]
        if s + NBUF - 1 < STEPS:
            nk = order[s + NBUF - 1]
            if nk == 0 and not drained_prompt[0]:
                drain_prompt_writes()
                drained_prompt[0] = True
            if pend_w[nk] is not None:
                pend_w[nk].wait()
                pend_w[nk] = None
            issue(s + NBUF - 1, nk)
        wait_gather(s, k)
        j, c = jcs[s]
        pend_w[k] = pltpu.async_copy(bufs[k], out_hbm.at[j, pl.ds(c * CHUNK, CHUNK)],
                                     sems_w[k])
    for k in range(NBUF):
        if pend_w[k] is not None:
            pend_w[k].wait()


def kernel(tokens, table, prompt_embedding):
    tok = jnp.transpose(tokens.astype(jnp.int32))
    sc = pl.kernel(
        _sc_body,
        out_type=jax.ShapeDtypeStruct((SEQ, BATCH, DIM), jnp.float32),
        mesh=plsc.VectorSubcoreMesh(core_axis_name="c", subcore_axis_name="s"),
        scratch_types=[
            pltpu.VMEM((STEPS * CHUNK,), jnp.int32),
            pltpu.VMEM((CHUNK, DIM), jnp.float32),
            pltpu.VMEM((CHUNK, DIM), jnp.float32),
            pltpu.VMEM((CHUNK, DIM), jnp.float32),
            pltpu.SemaphoreType.DMA,
            pltpu.SemaphoreType.DMA,
            pltpu.SemaphoreType.DMA,
            pltpu.SemaphoreType.DMA,
            pltpu.SemaphoreType.DMA,
            pltpu.SemaphoreType.DMA,
            pltpu.SemaphoreType.DMA,
        ],
    )
    out = sc(tok, table, prompt_embedding)
    return jnp.transpose(out, (1, 0, 2))


# speedup vs baseline: 1.9841x; 1.0033x over previous
import jax
import jax.numpy as jnp
from jax import lax
from jax.experimental import pallas as pl
from jax.experimental.pallas import tpu as pltpu
from jax.experimental.pallas import tpu_sc as plsc

VOCAB = 100000
DIM = 128
NUM_TOKENS = 20
BATCH = 1024
SEQ = 220
BODY = SEQ - NUM_TOKENS
CHUNK = 256
CPP = BATCH // CHUNK
NBODY = BODY * CPP
NBUF = 3

_info = plsc.get_sparse_core_info()
_NC, _NS = _info.num_cores, _info.num_subcores
NW = _NC * _NS
STEPS = NBODY // NW


def _sc_body(tok_hbm, table_hbm, prompt_hbm, out_hbm,
             idx_all, buf0, buf1, buf2,
             sem_i, sem_g0, sem_g1, sem_g2, sem_w0, sem_w1, sem_w2):
    wid = lax.axis_index("s") * _NC + lax.axis_index("c")

    bufs = (buf0, buf1, buf2)
    sems_g = (sem_g0, sem_g1, sem_g2)
    sems_w = (sem_w0, sem_w1, sem_w2)

    def chunk_of(s):
        t = wid + s * NW
        return NUM_TOKENS + t // CPP, t % CPP

    jcs = [chunk_of(s) for s in range(STEPS)]
    pend_i = []
    for s, (j, c) in enumerate(jcs):
        pend_i.append(pltpu.async_copy(
            tok_hbm.at[j, pl.ds(c * CHUNK, CHUNK)],
            idx_all.at[pl.ds(s * CHUNK, CHUNK)], sem_i))

    REP = 64

    @pl.when(wid < NUM_TOKENS)
    def _():
        pltpu.sync_copy(prompt_hbm.at[pl.ds(wid, 1)], buf0.at[pl.ds(0, 1)])
        vs = [buf0[0, pl.ds(q * 16, 16)] for q in range(DIM // 16)]

        def rep_body(r, carry):
            for q in range(DIM // 16):
                buf0[r, pl.ds(q * 16, 16)] = vs[q]
            return carry

        lax.fori_loop(1, REP, rep_body, 0)
        for h in range(BATCH // REP):
            pltpu.async_copy(buf0.at[pl.ds(0, REP)],
                             out_hbm.at[wid, pl.ds(h * REP, REP)], sem_w0)

    def drain_prompt_writes():
        for h in range(BATCH // REP):
            @pl.when(wid < NUM_TOKENS)
            def _(h=h):
                pltpu.make_async_copy(
                    buf0.at[pl.ds(0, REP)],
                    out_hbm.at[wid, pl.ds(h * REP, REP)], sem_w0
                ).wait()

    def issue(s, k):
        idx = idx_all.at[pl.ds(s * CHUNK, CHUNK)]
        pend_i[s].wait()
        pltpu.async_copy(table_hbm.at[idx], bufs[k], sems_g[k])

    def wait_gather(s, k):
        idx = idx_all.at[pl.ds(s * CHUNK, CHUNK)]
        pltpu.make_async_copy(table_hbm.at[idx], bufs[k], sems_g[k]).wait()

    order = [(s + 1) % NBUF for s in range(STEPS)]
    pend_w = [None] * NBUF
    issue(0, order[0])
    issue(1, order[1])
    drained_prompt = [False]
    for s in range(STEPS):
        k = order[s]
        if s + NBUF - 1 < STEPS:
            nk = order[s + NBUF - 1]
            if nk == 0 and not drained_prompt[0]:
                drain_prompt_writes()
                drained_prompt[0] = True
            if pend_w[nk] is not None:
                pend_w[nk].wait()
                pend_w[nk] = None
            issue(s + NBUF - 1, nk)
        wait_gather(s, k)
        j, c = jcs[s]
        pend_w[k] = pltpu.async_copy(bufs[k], out_hbm.at[j, pl.ds(c * CHUNK, CHUNK)],
                                     sems_w[k])
    for k in range(NBUF):
        if pend_w[k] is not None:
            pend_w[k].wait()


def kernel(tokens, table, prompt_embedding):
    tok = jnp.transpose(tokens.astype(jnp.int32))
    sc = pl.kernel(
        _sc_body,
        out_type=jax.ShapeDtypeStruct((SEQ, BATCH, DIM), jnp.float32),
        mesh=plsc.VectorSubcoreMesh(core_axis_name="c", subcore_axis_name="s"),
        scratch_types=[
            pltpu.VMEM((STEPS * CHUNK,), jnp.int32),
            pltpu.VMEM((CHUNK, DIM), jnp.float32),
            pltpu.VMEM((CHUNK, DIM), jnp.float32),
            pltpu.VMEM((CHUNK, DIM), jnp.float32),
            pltpu.SemaphoreType.DMA,
            pltpu.SemaphoreType.DMA,
            pltpu.SemaphoreType.DMA,
            pltpu.SemaphoreType.DMA,
            pltpu.SemaphoreType.DMA,
            pltpu.SemaphoreType.DMA,
            pltpu.SemaphoreType.DMA,
        ],
    )
    out = sc(tok, table, prompt_embedding)
    return jnp.transpose(out, (1, 0, 2))
